# baseline (speedup 1.0000x reference)
import functools

import jax
import jax.numpy as jnp
from jax import lax
from jax.experimental import pallas as pl
from jax.experimental.pallas import tpu as pltpu
from jax.experimental.pallas import tpu_sc as plsc

N = 10000
D = 128
E = 320000
ACTIVE = 64
CHANNEL = 16

NC = 2
NS = 16
DH = D // NC
K = 128
NBUF = 2
HCHUNKS = 80
CHUNKS = 2 * HCHUNKS
EPT = CHUNKS * K
E_PAD = NS * EPT
RPS = 632
N_PAD = 10016
RPS_LAST = N_PAD - (NS - 1) * RPS
TRS = 632
TRS_LAST = N - (NS - 1) * TRS

DO_GATHER = True
DO_SCATTER = True

SROWS = 79
ROWS_BLK = 1000

def _segsum_body(h_hbm, eidx_hbm, zero_hbm, out_hbm, agg_sp, tab_sp, ebuf,
                 *bufs):
  c = lax.axis_index("c")
  s = lax.axis_index("s")
  rbufs = bufs[:NBUF]
  gsems = bufs[NBUF:2 * NBUF]
  ssems = bufs[2 * NBUF:]

  def gather_start(ch, i, rb, sem):
    if i == 0:
      pltpu.async_copy(tab_sp.at[ebuf.at[ch, 0]], rb, sem)
    else:
      pltpu.async_copy(h_hbm.at[ebuf.at[ch, 0]], rb, sem)

  def gather_wait(ch, i, rb, sem):
    if i == 0:
      pltpu.make_async_copy(tab_sp.at[ebuf.at[ch, 0]], rb, sem).wait()
    else:
      pltpu.make_async_copy(h_hbm.at[ebuf.at[ch, 0]], rb, sem).wait()

  def scat_start(ch, rb, sem):
    pltpu.async_copy(rb, agg_sp.at[ebuf.at[ch, 1]], sem, add=True)

  def scat_wait(ch, rb, sem):
    pltpu.make_async_copy(rb, agg_sp.at[ebuf.at[ch, 1]], sem).wait()

  ebase = (c * NS + s) * CHUNKS
  pltpu.sync_copy(eidx_hbm.at[pl.ds(ebase, HCHUNKS)], ebuf)

  @pl.when(s < NS - 1)
  def _():
    pltpu.sync_copy(h_hbm.at[pl.ds(c * N + s * TRS, TRS)],
                    tab_sp.at[pl.ds(s * TRS, TRS)])
    pltpu.sync_copy(zero_hbm, agg_sp.at[pl.ds(s * RPS, RPS)])

  @pl.when(s == NS - 1)
  def _():
    pltpu.sync_copy(h_hbm.at[pl.ds(c * N + (NS - 1) * TRS, TRS_LAST)],
                    tab_sp.at[pl.ds((NS - 1) * TRS, TRS_LAST)])
    pltpu.sync_copy(zero_hbm.at[pl.ds(0, RPS_LAST)],
                    agg_sp.at[pl.ds((NS - 1) * RPS, RPS_LAST)])

  plsc.subcore_barrier()

  def run_half():
    for i in range(NBUF):
      gather_start(i, i, rbufs[i], gsems[i])

    def body(j, carry):
      for i in range(NBUF):
        ch = j * NBUF + i
        gather_wait(ch, i, rbufs[i], gsems[i])
        scat_start(ch, rbufs[i], ssems[i])
      for i in range(NBUF):
        ch = j * NBUF + i
        scat_wait(ch, rbufs[i], ssems[i])
        gather_start(ch + NBUF, i, rbufs[i], gsems[i])
      return carry

    lax.fori_loop(0, HCHUNKS // NBUF - 1, body, 0)
    last = HCHUNKS - NBUF
    for i in range(NBUF):
      gather_wait(last + i, i, rbufs[i], gsems[i])
      scat_start(last + i, rbufs[i], ssems[i])
    for i in range(NBUF):
      scat_wait(last + i, rbufs[i], ssems[i])

  run_half()
  pltpu.sync_copy(eidx_hbm.at[pl.ds(ebase + HCHUNKS, HCHUNKS)], ebuf)
  run_half()
  plsc.subcore_barrier()

  @pl.when(s < NS - 1)
  def _():
    pltpu.sync_copy(agg_sp.at[pl.ds(s * RPS, RPS)],
                    out_hbm.at[pl.ds(c * N_PAD + s * RPS, RPS)])

  @pl.when(s == NS - 1)
  def _():
    pltpu.sync_copy(agg_sp.at[pl.ds((NS - 1) * RPS, RPS_LAST)],
                    out_hbm.at[pl.ds(c * N_PAD + (NS - 1) * RPS, RPS_LAST)])


@functools.lru_cache(maxsize=1)
def _make_segsum():
  mesh = plsc.VectorSubcoreMesh(
      core_axis_name="c", subcore_axis_name="s", num_cores=NC, num_subcores=NS)
  return pl.kernel(
      _segsum_body,
      out_type=jax.ShapeDtypeStruct((NC * N_PAD, DH), jnp.float32),
      mesh=mesh,
      scratch_types=(
          [pltpu.VMEM_SHARED((N_PAD, DH), jnp.float32),
           pltpu.VMEM_SHARED((N, DH), jnp.float32),
           pltpu.VMEM((HCHUNKS, 2, K), jnp.int32)]
          + [pltpu.VMEM((K, DH), jnp.float32)] * NBUF
          + [pltpu.SemaphoreType.DMA] * (2 * NBUF)
      ),
      compiler_params=pltpu.CompilerParams(use_tc_tiling_on_sc=False),
  )


def _segsum(h, eidx, zero_rows):
  return _make_segsum()(h, eidx, zero_rows)


def _gin_dense_body(scale_ref, ab_ref, h_ref, a_ref, w_ref, b_ref,
                    aw_ref, hout_ref, score_ref):
  t = (jnp.concatenate([h_ref[0], h_ref[1]], axis=1) * scale_ref[0]
       + jnp.concatenate([a_ref[0], a_ref[1]], axis=1))
  hh = jnp.dot(t, w_ref[...], preferred_element_type=jnp.float32) + b_ref[...]
  hh = jnp.maximum(hh, 0.0)
  hout_ref[0] = hh[:, :DH]
  hout_ref[1] = hh[:, DH:]
  sc = jnp.dot(hh, aw_ref[...], preferred_element_type=jnp.float32) + ab_ref[0]
  score_ref[...] = jax.nn.sigmoid(sc)


def _gin_dense(h2, agg2, W, b, scale, aw, ab):
  nblk = N // ROWS_BLK
  return pl.pallas_call(
      _gin_dense_body,
      grid=(nblk,),
      in_specs=[
          pl.BlockSpec(memory_space=pltpu.SMEM),
          pl.BlockSpec(memory_space=pltpu.SMEM),
          pl.BlockSpec((NC, ROWS_BLK, DH), lambda i: (0, i, 0)),
          pl.BlockSpec((NC, ROWS_BLK, DH), lambda i: (0, i, 0)),
          pl.BlockSpec((D, D), lambda i: (0, 0)),
          pl.BlockSpec((1, D), lambda i: (0, 0)),
          pl.BlockSpec((D, 1), lambda i: (0, 0)),
      ],
      out_specs=[
          pl.BlockSpec((NC, ROWS_BLK, DH), lambda i: (0, i, 0)),
          pl.BlockSpec((ROWS_BLK, 1), lambda i: (i, 0)),
      ],
      out_shape=[
          jax.ShapeDtypeStruct((NC, N, DH), jnp.float32),
          jax.ShapeDtypeStruct((N, 1), jnp.float32),
      ],
  )(scale, ab, h2, agg2.reshape(NC, N_PAD, DH), W, b.reshape(1, D), aw)


def _pool_head_body(h1_ref, h2_ref, h3_ref, s1_ref, s2_ref, s3_ref,
                    convw_ref, convb_ref, fc1w_ref, fc1b_ref, fc2w_ref,
                    fc2b_ref, out_ref, sbuf):
  lin = (lax.broadcasted_iota(jnp.int32, (SROWS, 128), 0) * 128 +
         lax.broadcasted_iota(jnp.int32, (SROWS, 128), 1))
  col_iota = lax.broadcasted_iota(jnp.int32, (ACTIVE, N), 1)
  t_iota = lax.broadcasted_iota(jnp.int32, (ACTIVE, 1), 0)

  sbuf[0] = s1_ref[...]
  sbuf[1] = s2_ref[...]
  sbuf[2] = s3_ref[...]

  def body(t, carry):
    new_carry = []
    for l in range(3):
      idx_vec, m_vec = carry[l]
      sv = sbuf[l]
      m = jnp.max(sv)
      idx = jnp.min(jnp.where(sv == m, lin, jnp.int32(1 << 30)))
      idx_vec = jnp.where(t_iota == t, idx, idx_vec)
      m_vec = jnp.where(t_iota == t, m, m_vec)
      sbuf[l] = jnp.where(lin == idx, -1.0, sv)
      new_carry.append((idx_vec, m_vec))
    return tuple(new_carry)

  init = tuple(
      (jnp.zeros((ACTIVE, 1), jnp.int32), jnp.zeros((ACTIVE, 1), jnp.float32))
      for _ in range(3))
  sel = lax.fori_loop(0, ACTIVE, body, init)

  conv = convb_ref[...]
  for l, h_ref in enumerate((h1_ref, h2_ref, h3_ref)):
    idx_vec, m_vec = sel[l]
    oh = jnp.where(col_iota == idx_vec, 1.0, 0.0)
    wrow = jnp.concatenate(
        [lax.dot_general(oh, h_ref[0], (((1,), (0,)), ((), ())),
                         preferred_element_type=jnp.float32),
         lax.dot_general(oh, h_ref[1], (((1,), (0,)), ((), ())),
                         preferred_element_type=jnp.float32)], axis=1) * m_vec
    cw = convw_ref[...][:, l * D:(l + 1) * D]
    conv = conv + lax.dot_general(cw, wrow, (((1,), (1,)), ((), ())),
                                  preferred_element_type=jnp.float32)
  conv = jnp.maximum(conv, 0.0)
  acc = jnp.zeros((1, 128), jnp.float32)
  for ch in range(CHANNEL):
    acc = acc + jnp.dot(conv[ch:ch + 1, :], fc1w_ref[ch],
                        preferred_element_type=jnp.float32)
  f1 = jnp.maximum(acc + fc1b_ref[...], 0.0)
  f2 = jnp.dot(f1, fc2w_ref[...], preferred_element_type=jnp.float32)
  out_ref[...] = jax.nn.sigmoid(f2 + fc2b_ref[...])


_pool_head = pl.pallas_call(
    _pool_head_body,
    out_shape=jax.ShapeDtypeStruct((1, 128), jnp.float32),
    scratch_shapes=[
        pltpu.VMEM((3, SROWS, 128), jnp.float32),
    ],
)


def _pad_scores(s):
  return jnp.concatenate(
      [s[:, 0], jnp.full((SROWS * 128 - N,), -1.0, jnp.float32)]
  ).reshape(SROWS, 128)


def kernel(x, edge_index, W1, b1, eps1, attw1, attb1, W2, b2, eps2, attw2,
           attb2, W3, b3, eps3, attw3, attb3, conv_w, conv_b, fc1_w, fc1_b,
           fc2_w, fc2_b):
  src = jnp.concatenate(
      [edge_index[0], jnp.zeros((E_PAD - E,), jnp.int32)])
  dst = jnp.concatenate(
      [edge_index[1], jnp.full((E_PAD - E,), N, jnp.int32)])
  srcw = src.reshape(NS, CHUNKS, K)
  dstw = dst.reshape(NS, CHUNKS, K)
  hbm_off = (jnp.arange(CHUNKS, dtype=jnp.int32)[None, :, None] % 2) * N
  eidx = jnp.concatenate(
      [jnp.stack([srcw, dstw], axis=2),
       jnp.stack([srcw + hbm_off, dstw], axis=2)]
  ).reshape(NC * NS * CHUNKS, 2, K)
  zero_rows = jnp.zeros((RPS, DH), jnp.float32)

  h = jnp.concatenate([x[:, :DH], x[:, DH:]], axis=0).reshape(NC, N, DH)
  scores = []
  hs = []
  for W, b, eps, aw, ab in ((W1, b1, eps1, attw1, attb1),
                            (W2, b2, eps2, attw2, attb2),
                            (W3, b3, eps3, attw3, attb3)):
    agg2 = _segsum(h.reshape(NC * N, DH), eidx, zero_rows)
    h, s = _gin_dense(h, agg2, W, b,
                      (1.0 + eps).reshape(1), aw, ab.reshape(1))
    hs.append(h)
    scores.append(_pad_scores(s))

  fc2_w_pad = jnp.concatenate(
      [fc2_w, jnp.zeros((128, 126), jnp.float32)], axis=1)
  fc2_b_pad = jnp.concatenate(
      [fc2_b, jnp.zeros((126,), jnp.float32)]).reshape(1, 128)
  out = _pool_head(hs[0], hs[1], hs[2], scores[0], scores[1], scores[2],
                   conv_w, conv_b.reshape(CHANNEL, 1),
                   fc1_w.reshape(CHANNEL, ACTIVE, 128), fc1_b.reshape(1, 128),
                   fc2_w_pad, fc2_b_pad)
  return out[:, :2]


# speedup vs baseline: 1.3124x; 1.3124x over previous
import functools

import jax
import jax.numpy as jnp
from jax import lax
from jax.experimental import pallas as pl
from jax.experimental.pallas import tpu as pltpu
from jax.experimental.pallas import tpu_sc as plsc

N = 10000
D = 128
E = 320000
ACTIVE = 64
CHANNEL = 16

NC = 2
NS = 16
DH = D // NC
K = 128
NBUF = 2
HCHUNKS = 80
CHUNKS = 2 * HCHUNKS
EPT = CHUNKS * K
E_PAD = NS * EPT
RPS = 632
N_PAD = 10016
RPS_LAST = N_PAD - (NS - 1) * RPS
TRS = 632
TRS_LAST = N - (NS - 1) * TRS

DO_GATHER = True
DO_SCATTER = True

SROWS = 79
ROWS_BLK = 1000

def _segsum_body(h_hbm, eidx_hbm, zero_hbm, out_hbm, agg_sp, tab_sp, ebuf,
                 *bufs):
  c = lax.axis_index("c")
  s = lax.axis_index("s")
  rbufs = bufs[:NBUF]
  gsems = bufs[NBUF:2 * NBUF]
  ssems = bufs[2 * NBUF:]

  def gather_start(ch, i, rb, sem):
    pltpu.async_copy(tab_sp.at[ebuf.at[ch, 0]], rb, sem)

  def gather_wait(ch, i, rb, sem):
    pltpu.make_async_copy(tab_sp.at[ebuf.at[ch, 0]], rb, sem).wait()

  def scat_start(ch, rb, sem):
    pltpu.async_copy(rb, agg_sp.at[ebuf.at[ch, 1]], sem, add=True)

  def scat_wait(ch, rb, sem):
    pltpu.make_async_copy(rb, agg_sp.at[ebuf.at[ch, 1]], sem).wait()

  ebase = s * CHUNKS
  pltpu.sync_copy(eidx_hbm.at[pl.ds(ebase, HCHUNKS)], ebuf)

  @pl.when(s < NS - 1)
  def _():
    pltpu.sync_copy(h_hbm.at[pl.ds(c * N + s * TRS, TRS)],
                    tab_sp.at[pl.ds(s * TRS, TRS)])
    pltpu.sync_copy(zero_hbm, agg_sp.at[pl.ds(s * RPS, RPS)])

  @pl.when(s == NS - 1)
  def _():
    pltpu.sync_copy(h_hbm.at[pl.ds(c * N + (NS - 1) * TRS, TRS_LAST)],
                    tab_sp.at[pl.ds((NS - 1) * TRS, TRS_LAST)])
    pltpu.sync_copy(zero_hbm.at[pl.ds(0, RPS_LAST)],
                    agg_sp.at[pl.ds((NS - 1) * RPS, RPS_LAST)])

  plsc.subcore_barrier()

  def run_half():
    for i in range(NBUF):
      gather_start(i, i, rbufs[i], gsems[i])

    def body(j, carry):
      for i in range(NBUF):
        ch = j * NBUF + i
        gather_wait(ch, i, rbufs[i], gsems[i])
        scat_start(ch, rbufs[i], ssems[i])
      for i in range(NBUF):
        ch = j * NBUF + i
        scat_wait(ch, rbufs[i], ssems[i])
        gather_start(ch + NBUF, i, rbufs[i], gsems[i])
      return carry

    lax.fori_loop(0, HCHUNKS // NBUF - 1, body, 0)
    last = HCHUNKS - NBUF
    for i in range(NBUF):
      gather_wait(last + i, i, rbufs[i], gsems[i])
      scat_start(last + i, rbufs[i], ssems[i])
    for i in range(NBUF):
      scat_wait(last + i, rbufs[i], ssems[i])

  run_half()
  pltpu.sync_copy(eidx_hbm.at[pl.ds(ebase + HCHUNKS, HCHUNKS)], ebuf)
  run_half()
  plsc.subcore_barrier()

  @pl.when(s < NS - 1)
  def _():
    pltpu.sync_copy(agg_sp.at[pl.ds(s * RPS, RPS)],
                    out_hbm.at[pl.ds(c * N_PAD + s * RPS, RPS)])

  @pl.when(s == NS - 1)
  def _():
    pltpu.sync_copy(agg_sp.at[pl.ds((NS - 1) * RPS, RPS_LAST)],
                    out_hbm.at[pl.ds(c * N_PAD + (NS - 1) * RPS, RPS_LAST)])


@functools.lru_cache(maxsize=1)
def _make_segsum():
  mesh = plsc.VectorSubcoreMesh(
      core_axis_name="c", subcore_axis_name="s", num_cores=NC, num_subcores=NS)
  return pl.kernel(
      _segsum_body,
      out_type=jax.ShapeDtypeStruct((NC * N_PAD, DH), jnp.float32),
      mesh=mesh,
      scratch_types=(
          [pltpu.VMEM_SHARED((N_PAD, DH), jnp.float32),
           pltpu.VMEM_SHARED((N, DH), jnp.float32),
           pltpu.VMEM((HCHUNKS, 2, K), jnp.int32)]
          + [pltpu.VMEM((K, DH), jnp.float32)] * NBUF
          + [pltpu.SemaphoreType.DMA] * (2 * NBUF)
      ),
      compiler_params=pltpu.CompilerParams(use_tc_tiling_on_sc=False),
  )


def _segsum(h, eidx, zero_rows):
  return _make_segsum()(h, eidx, zero_rows)


def _gin_dense_body(scale_ref, ab_ref, h_ref, a_ref, w_ref, b_ref,
                    aw_ref, hout_ref, score_ref):
  t = (jnp.concatenate([h_ref[0], h_ref[1]], axis=1) * scale_ref[0]
       + jnp.concatenate([a_ref[0], a_ref[1]], axis=1))
  hh = jnp.dot(t, w_ref[...], preferred_element_type=jnp.float32) + b_ref[...]
  hh = jnp.maximum(hh, 0.0)
  hout_ref[0] = hh[:, :DH]
  hout_ref[1] = hh[:, DH:]
  sc = jnp.dot(hh, aw_ref[...], preferred_element_type=jnp.float32) + ab_ref[0]
  score_ref[...] = jax.nn.sigmoid(sc)


def _gin_dense(h2, agg2, W, b, scale, aw, ab):
  nblk = N // ROWS_BLK
  return pl.pallas_call(
      _gin_dense_body,
      grid=(nblk,),
      in_specs=[
          pl.BlockSpec(memory_space=pltpu.SMEM),
          pl.BlockSpec(memory_space=pltpu.SMEM),
          pl.BlockSpec((NC, ROWS_BLK, DH), lambda i: (0, i, 0)),
          pl.BlockSpec((NC, ROWS_BLK, DH), lambda i: (0, i, 0)),
          pl.BlockSpec((D, D), lambda i: (0, 0)),
          pl.BlockSpec((1, D), lambda i: (0, 0)),
          pl.BlockSpec((D, 1), lambda i: (0, 0)),
      ],
      out_specs=[
          pl.BlockSpec((NC, ROWS_BLK, DH), lambda i: (0, i, 0)),
          pl.BlockSpec((ROWS_BLK, 1), lambda i: (i, 0)),
      ],
      out_shape=[
          jax.ShapeDtypeStruct((NC, N, DH), jnp.float32),
          jax.ShapeDtypeStruct((N, 1), jnp.float32),
      ],
  )(scale, ab, h2, agg2.reshape(NC, N_PAD, DH), W, b.reshape(1, D), aw)


def _pool_head_body(h1_ref, h2_ref, h3_ref, s1_ref, s2_ref, s3_ref,
                    convw_ref, convb_ref, fc1w_ref, fc1b_ref, fc2w_ref,
                    fc2b_ref, out_ref, sbuf):
  lin = (lax.broadcasted_iota(jnp.int32, (SROWS, 128), 0) * 128 +
         lax.broadcasted_iota(jnp.int32, (SROWS, 128), 1))
  col_iota = lax.broadcasted_iota(jnp.int32, (ACTIVE, N), 1)
  t_iota = lax.broadcasted_iota(jnp.int32, (ACTIVE, 1), 0)

  sbuf[0] = s1_ref[...]
  sbuf[1] = s2_ref[...]
  sbuf[2] = s3_ref[...]

  def body(t, carry):
    new_carry = []
    for l in range(3):
      idx_vec, m_vec = carry[l]
      sv = sbuf[l]
      m = jnp.max(sv)
      idx = jnp.min(jnp.where(sv == m, lin, jnp.int32(1 << 30)))
      idx_vec = jnp.where(t_iota == t, idx, idx_vec)
      m_vec = jnp.where(t_iota == t, m, m_vec)
      sbuf[l] = jnp.where(lin == idx, -1.0, sv)
      new_carry.append((idx_vec, m_vec))
    return tuple(new_carry)

  init = tuple(
      (jnp.zeros((ACTIVE, 1), jnp.int32), jnp.zeros((ACTIVE, 1), jnp.float32))
      for _ in range(3))
  sel = lax.fori_loop(0, ACTIVE, body, init)

  conv = convb_ref[...]
  for l, h_ref in enumerate((h1_ref, h2_ref, h3_ref)):
    idx_vec, m_vec = sel[l]
    oh = jnp.where(col_iota == idx_vec, 1.0, 0.0)
    wrow = jnp.concatenate(
        [lax.dot_general(oh, h_ref[0], (((1,), (0,)), ((), ())),
                         preferred_element_type=jnp.float32),
         lax.dot_general(oh, h_ref[1], (((1,), (0,)), ((), ())),
                         preferred_element_type=jnp.float32)], axis=1) * m_vec
    cw = convw_ref[...][:, l * D:(l + 1) * D]
    conv = conv + lax.dot_general(cw, wrow, (((1,), (1,)), ((), ())),
                                  preferred_element_type=jnp.float32)
  conv = jnp.maximum(conv, 0.0)
  acc = jnp.zeros((1, 128), jnp.float32)
  for ch in range(CHANNEL):
    acc = acc + jnp.dot(conv[ch:ch + 1, :], fc1w_ref[ch],
                        preferred_element_type=jnp.float32)
  f1 = jnp.maximum(acc + fc1b_ref[...], 0.0)
  f2 = jnp.dot(f1, fc2w_ref[...], preferred_element_type=jnp.float32)
  out_ref[...] = jax.nn.sigmoid(f2 + fc2b_ref[...])


_pool_head = pl.pallas_call(
    _pool_head_body,
    out_shape=jax.ShapeDtypeStruct((1, 128), jnp.float32),
    scratch_shapes=[
        pltpu.VMEM((3, SROWS, 128), jnp.float32),
    ],
)


def _pad_scores(s):
  return jnp.concatenate(
      [s[:, 0], jnp.full((SROWS * 128 - N,), -1.0, jnp.float32)]
  ).reshape(SROWS, 128)


def kernel(x, edge_index, W1, b1, eps1, attw1, attb1, W2, b2, eps2, attw2,
           attb2, W3, b3, eps3, attw3, attb3, conv_w, conv_b, fc1_w, fc1_b,
           fc2_w, fc2_b):
  src = jnp.concatenate(
      [edge_index[0], jnp.zeros((E_PAD - E,), jnp.int32)])
  dst = jnp.concatenate(
      [edge_index[1], jnp.full((E_PAD - E,), N, jnp.int32)])
  srcw = src.reshape(NS, CHUNKS, K)
  dstw = dst.reshape(NS, CHUNKS, K)
  eidx = jnp.stack([srcw, dstw], axis=2).reshape(NS * CHUNKS, 2, K)
  zero_rows = jnp.zeros((RPS, DH), jnp.float32)

  h = jnp.concatenate([x[:, :DH], x[:, DH:]], axis=0).reshape(NC, N, DH)
  scores = []
  hs = []
  for W, b, eps, aw, ab in ((W1, b1, eps1, attw1, attb1),
                            (W2, b2, eps2, attw2, attb2),
                            (W3, b3, eps3, attw3, attb3)):
    agg2 = _segsum(h.reshape(NC * N, DH), eidx, zero_rows)
    h, s = _gin_dense(h, agg2, W, b,
                      (1.0 + eps).reshape(1), aw, ab.reshape(1))
    hs.append(h)
    scores.append(_pad_scores(s))

  fc2_w_pad = jnp.concatenate(
      [fc2_w, jnp.zeros((128, 126), jnp.float32)], axis=1)
  fc2_b_pad = jnp.concatenate(
      [fc2_b, jnp.zeros((126,), jnp.float32)]).reshape(1, 128)
  out = _pool_head(hs[0], hs[1], hs[2], scores[0], scores[1], scores[2],
                   conv_w, conv_b.reshape(CHANNEL, 1),
                   fc1_w.reshape(CHANNEL, ACTIVE, 128), fc1_b.reshape(1, 128),
                   fc2_w_pad, fc2_b_pad)
  return out[:, :2]
